# baseline (speedup 1.0000x reference)
import functools

import jax
import jax.numpy as jnp
from jax import lax
from jax.experimental import pallas as pl
from jax.experimental.pallas import tpu as pltpu
from jax.experimental.pallas import tpu_sc as plsc

N = 10000
NP = 10240
E = 160000
D = 256
CLS = 64
NC, NS = 2, 16
CHUNK = 128
E_PAD = 163840
ROWS_T = NP // NS

_mesh = plsc.VectorSubcoreMesh(core_axis_name="c", subcore_axis_name="s")


def _make_deg_kernel():
    cpt = E_PAD // (NC * NS * CHUNK)

    @functools.partial(
        pl.kernel,
        out_type=jax.ShapeDtypeStruct((2 * NP, 128), jnp.float32),
        mesh=_mesh,
        scratch_types=[
            pltpu.VMEM_SHARED((NP, 128), jnp.float32),
            pltpu.VMEM((cpt, CHUNK), jnp.int32),
            pltpu.VMEM((CHUNK, 128), jnp.float32),
        ],
    )
    def deg_kernel(dst_hbm, zeros_hbm, ones_hbm, deg_out, acc, d_idx, ones_v):
        c = lax.axis_index("c")
        s = lax.axis_index("s")
        pltpu.sync_copy(zeros_hbm.at[pl.ds(s * ROWS_T, ROWS_T)],
                        acc.at[pl.ds(s * ROWS_T, ROWS_T)])
        pltpu.sync_copy(ones_hbm, ones_v)
        base = c * (E_PAD // (NC * CHUNK)) + s * cpt
        pltpu.sync_copy(dst_hbm.at[pl.ds(base, cpt)], d_idx)
        plsc.subcore_barrier()

        def body(j, carry):
            pltpu.sync_copy(ones_v, acc.at[d_idx.at[j]], add=True)
            return carry

        lax.fori_loop(0, cpt, body, 0)
        plsc.subcore_barrier()
        pltpu.sync_copy(acc.at[pl.ds(s * ROWS_T, ROWS_T)],
                        deg_out.at[pl.ds(c * NP + s * ROWS_T, ROWS_T)])

    return deg_kernel


_IDXG = 8


def _edge_pipeline(g_hbm, acc, src_hbm, dst_hbm, sbase, dbase,
                   s_idx, d_idx, rows_a, rows_b, sem_a, sem_b, cpt):
    def group(gi, carry):
        pltpu.sync_copy(src_hbm.at[pl.ds(sbase + gi * _IDXG, _IDXG)], s_idx)
        pltpu.sync_copy(dst_hbm.at[pl.ds(dbase + gi * _IDXG, _IDXG)], d_idx)
        pltpu.async_copy(g_hbm.at[s_idx.at[0]], rows_a, sem_a)

        def body(i, c2):
            e0 = 2 * i
            e1 = e0 + 1
            pltpu.async_copy(g_hbm.at[s_idx.at[e1]], rows_b, sem_b)
            pltpu.make_async_copy(g_hbm.at[s_idx.at[e0]], rows_a,
                                  sem_a).wait()
            pltpu.sync_copy(rows_a, acc.at[d_idx.at[e0]], add=True)

            @pl.when(e1 + 1 < _IDXG)
            def _():
                pltpu.async_copy(g_hbm.at[s_idx.at[e1 + 1]], rows_a, sem_a)

            pltpu.make_async_copy(g_hbm.at[s_idx.at[e1]], rows_b,
                                  sem_b).wait()
            pltpu.sync_copy(rows_b, acc.at[d_idx.at[e1]], add=True)
            return c2

        lax.fori_loop(0, _IDXG // 2, body, 0)
        return carry

    lax.fori_loop(0, cpt // _IDXG, group, 0)


def _make_agg_kernel(dh):
    cpt = E_PAD // (NS * CHUNK)

    @functools.partial(
        pl.kernel,
        out_type=jax.ShapeDtypeStruct((2 * NP, dh), jnp.float32),
        mesh=_mesh,
        scratch_types=[
            pltpu.VMEM_SHARED((NP, dh), jnp.float32),
            pltpu.VMEM((_IDXG, CHUNK), jnp.int32),
            pltpu.VMEM((_IDXG, CHUNK), jnp.int32),
            pltpu.VMEM((CHUNK, dh), jnp.float32),
            pltpu.VMEM((CHUNK, dh), jnp.float32),
            pltpu.SemaphoreType.DMA,
            pltpu.SemaphoreType.DMA,
        ],
    )
    def agg_kernel(g_hbm, src_hbm, dst_hbm, u_out, acc, s_idx, d_idx,
                   rows_a, rows_b, sem_a, sem_b):
        c = lax.axis_index("c")
        s = lax.axis_index("s")
        pltpu.sync_copy(g_hbm.at[pl.ds(c * NP + s * ROWS_T, ROWS_T)],
                        acc.at[pl.ds(s * ROWS_T, ROWS_T)])
        plsc.subcore_barrier()
        _edge_pipeline(g_hbm, acc, src_hbm, dst_hbm,
                       c * (cpt * NS) + s * cpt, s * cpt,
                       s_idx, d_idx, rows_a, rows_b, sem_a, sem_b, cpt)
        plsc.subcore_barrier()
        pltpu.sync_copy(acc.at[pl.ds(s * ROWS_T, ROWS_T)],
                        u_out.at[pl.ds(c * NP + s * ROWS_T, ROWS_T)])

    return agg_kernel


def _make_agg3_kernel():
    cpt = E_PAD // (NC * NS * CHUNK)

    @functools.partial(
        pl.kernel,
        out_type=jax.ShapeDtypeStruct((2 * NP, 128), jnp.float32),
        mesh=_mesh,
        scratch_types=[
            pltpu.VMEM_SHARED((NP, 128), jnp.float32),
            pltpu.VMEM((_IDXG, CHUNK), jnp.int32),
            pltpu.VMEM((_IDXG, CHUNK), jnp.int32),
            pltpu.VMEM((CHUNK, 128), jnp.float32),
            pltpu.VMEM((CHUNK, 128), jnp.float32),
            pltpu.SemaphoreType.DMA,
            pltpu.SemaphoreType.DMA,
        ],
    )
    def agg3_kernel(g_hbm, zeros_hbm, src_hbm, dst_hbm, u_out,
                    acc, s_idx, d_idx, rows_a, rows_b, sem_a, sem_b):
        c = lax.axis_index("c")
        s = lax.axis_index("s")
        pltpu.sync_copy(zeros_hbm.at[pl.ds(s * ROWS_T, ROWS_T)],
                        acc.at[pl.ds(s * ROWS_T, ROWS_T)])
        base = c * (E_PAD // (NC * CHUNK)) + s * cpt
        plsc.subcore_barrier()
        _edge_pipeline(g_hbm, acc, src_hbm, dst_hbm, base, base,
                       s_idx, d_idx, rows_a, rows_b, sem_a, sem_b, cpt)
        plsc.subcore_barrier()
        pltpu.sync_copy(acc.at[pl.ds(s * ROWS_T, ROWS_T)],
                        u_out.at[pl.ds(c * NP + s * ROWS_T, ROWS_T)])

    return agg3_kernel


_deg_kernel = _make_deg_kernel()
_agg128 = _make_agg_kernel(128)
_agg3 = _make_agg3_kernel()


_BM = 640
_GRID = NP // _BM


def _elu(v):
    return jnp.where(v > 0, v, jnp.exp(jnp.minimum(v, 0.0)) - 1.0)


def _tc1a_body(x_ref, w_ref, h_ref):
    h_ref[...] = jnp.dot(x_ref[...], w_ref[...],
                         preferred_element_type=jnp.float32)


def _tc1b_body(h_ref, deg0_ref, deg1_ref, g_ref, dinv_ref):
    deg = deg0_ref[:, 0] + deg1_ref[:, 0] + 1.0
    dinv = lax.rsqrt(deg)[:, None]
    dinv_ref[...] = jnp.broadcast_to(dinv, dinv_ref.shape)
    g = h_ref[...] * dinv
    g_ref[0] = g[:, :128]
    g_ref[1] = g[:, 128:]


def _tc2_body(u_ref, dinv_ref, b_ref, w_ref, x1_ref, g_ref):
    dinv = dinv_ref[:, :1]
    u = jnp.concatenate([u_ref[0], u_ref[1]], axis=1)
    x1 = _elu(dinv * u + b_ref[...])
    x1_ref[...] = x1
    g = jnp.dot(x1, w_ref[...], preferred_element_type=jnp.float32) * dinv
    g_ref[0] = g[:, :128]
    g_ref[1] = g[:, 128:]


def _tc3_body(u_ref, dinv_ref, b_ref, x1_ref, w_ref, g_ref):
    dinv = dinv_ref[:, :1]
    u = jnp.concatenate([u_ref[0], u_ref[1]], axis=1)
    x2 = _elu(dinv * u + b_ref[...] + x1_ref[...])
    g = jnp.dot(x2, w_ref[...], preferred_element_type=jnp.float32) * dinv
    g_ref[...] = jnp.concatenate([g, jnp.zeros_like(g)], axis=1)


def _tc4_body(u_ref, g_ref, dinv_ref, b_ref, o_ref):
    u = u_ref[0, :, :CLS] + u_ref[1, :, :CLS] + g_ref[:, :CLS]
    o_ref[...] = dinv_ref[:, :1] * u + b_ref[...]


def _row_spec(cols):
    return pl.BlockSpec((_BM, cols), lambda i: (i, 0))


def _half_spec(cols):
    return pl.BlockSpec((2, _BM, cols), lambda i: (0, i, 0))


def _full_spec(shape):
    return pl.BlockSpec(shape, lambda i: tuple(0 for _ in shape))


_tc1a = pl.pallas_call(
    _tc1a_body,
    grid=(_GRID,),
    in_specs=[_row_spec(D), _full_spec((D, D))],
    out_specs=_row_spec(D),
    out_shape=jax.ShapeDtypeStruct((NP, D), jnp.float32),
)

_tc1b = pl.pallas_call(
    _tc1b_body,
    grid=(_GRID,),
    in_specs=[_row_spec(D), _row_spec(128),
              pl.BlockSpec((_BM, 128), lambda i: (_GRID + i, 0))],
    out_specs=[_half_spec(128), _row_spec(128)],
    out_shape=[jax.ShapeDtypeStruct((2, NP, 128), jnp.float32),
               jax.ShapeDtypeStruct((NP, 128), jnp.float32)],
)

_tc2 = pl.pallas_call(
    _tc2_body,
    grid=(_GRID,),
    in_specs=[_half_spec(128), _row_spec(128), _full_spec((1, D)),
              _full_spec((D, D))],
    out_specs=[_row_spec(D), _half_spec(128)],
    out_shape=[jax.ShapeDtypeStruct((NP, D), jnp.float32),
               jax.ShapeDtypeStruct((2, NP, 128), jnp.float32)],
)

_tc3 = pl.pallas_call(
    _tc3_body,
    grid=(_GRID,),
    in_specs=[_half_spec(128), _row_spec(128), _full_spec((1, D)),
              _row_spec(D), _full_spec((D, CLS))],
    out_specs=_row_spec(128),
    out_shape=jax.ShapeDtypeStruct((NP, 128), jnp.float32),
)

_tc4 = pl.pallas_call(
    _tc4_body,
    grid=(_GRID,),
    in_specs=[_half_spec(128), _row_spec(128), _row_spec(128),
              _full_spec((1, CLS))],
    out_specs=_row_spec(CLS),
    out_shape=jax.ShapeDtypeStruct((NP, CLS), jnp.float32),
)


@jax.jit
def kernel(x, edge_index, batch, W1, b1, W2, b2, W3, b3):
    del batch
    src = edge_index[0]
    dst = edge_index[1]
    pad = E_PAD - E
    srcp = jnp.concatenate([src, jnp.zeros((pad,), jnp.int32)])
    dstp = jnp.concatenate([dst, jnp.full((pad,), N, jnp.int32)])
    src2 = jnp.stack([srcp, srcp + NP]).reshape(2 * (E_PAD // CHUNK), CHUNK)
    dstm = dstp.reshape(E_PAD // CHUNK, CHUNK)
    xp = jnp.pad(x, ((0, NP - N), (0, 0)))

    degf = _deg_kernel(dstm, jnp.zeros((NP, 128), jnp.float32),
                       jnp.ones((CHUNK, 128), jnp.float32))
    h1 = _tc1a(xp, W1)
    g1, dinvb = _tc1b(h1, degf, degf)
    u1 = _agg128(g1.reshape(2 * NP, 128), src2, dstm)
    x1, g2 = _tc2(u1.reshape(2, NP, 128), dinvb, b1.reshape(1, D), W2)
    u2 = _agg128(g2.reshape(2 * NP, 128), src2, dstm)
    g3 = _tc3(u2.reshape(2, NP, 128), dinvb, b2.reshape(1, D), x1, W3)
    u3 = _agg3(g3, jnp.zeros((NP, 128), jnp.float32),
               srcp.reshape(E_PAD // CHUNK, CHUNK), dstm)
    out = _tc4(u3.reshape(2, NP, 128), g3, dinvb, b3.reshape(1, CLS))
    return out[:N]


# speedup vs baseline: 1.1329x; 1.1329x over previous
import functools

import jax
import jax.numpy as jnp
from jax import lax
from jax.experimental import pallas as pl
from jax.experimental.pallas import tpu as pltpu
from jax.experimental.pallas import tpu_sc as plsc

N = 10000
NP = 10240
E = 160000
D = 256
CLS = 64
NC, NS = 2, 16
CHUNK = 128
E_PAD = 163840
ROWS_T = NP // NS

_mesh = plsc.VectorSubcoreMesh(core_axis_name="c", subcore_axis_name="s")


def _make_deg_kernel():
    cpt = E_PAD // (NC * NS * CHUNK)

    @functools.partial(
        pl.kernel,
        out_type=jax.ShapeDtypeStruct((2 * NP, 128), jnp.float32),
        mesh=_mesh,
        scratch_types=[
            pltpu.VMEM_SHARED((NP, 128), jnp.float32),
            pltpu.VMEM((cpt, CHUNK), jnp.int32),
            pltpu.VMEM((CHUNK, 128), jnp.float32),
        ],
    )
    def deg_kernel(dst_hbm, zeros_hbm, ones_hbm, deg_out, acc, d_idx, ones_v):
        c = lax.axis_index("c")
        s = lax.axis_index("s")
        pltpu.sync_copy(zeros_hbm.at[pl.ds(s * ROWS_T, ROWS_T)],
                        acc.at[pl.ds(s * ROWS_T, ROWS_T)])
        pltpu.sync_copy(ones_hbm, ones_v)
        base = c * (E_PAD // (NC * CHUNK)) + s * cpt
        pltpu.sync_copy(dst_hbm.at[pl.ds(base, cpt)], d_idx)
        plsc.subcore_barrier()

        def body(j, carry):
            pltpu.sync_copy(ones_v, acc.at[d_idx.at[j]], add=True)
            return carry

        lax.fori_loop(0, cpt, body, 0)
        plsc.subcore_barrier()
        pltpu.sync_copy(acc.at[pl.ds(s * ROWS_T, ROWS_T)],
                        deg_out.at[pl.ds(c * NP + s * ROWS_T, ROWS_T)])

    return deg_kernel


_IDXG = 8


def _edge_pipeline(g_hbm, acc, src_hbm, dst_hbm, sbase, dbase,
                   s_idx, d_idx, rows_a, rows_b, sem_a, sem_b, cpt):
    def group(gi, carry):
        pltpu.sync_copy(src_hbm.at[pl.ds(sbase + gi * _IDXG, _IDXG)], s_idx)
        pltpu.sync_copy(dst_hbm.at[pl.ds(dbase + gi * _IDXG, _IDXG)], d_idx)
        pltpu.async_copy(g_hbm.at[s_idx.at[0]], rows_a, sem_a)

        def body(i, c2):
            e0 = 2 * i
            e1 = e0 + 1
            pltpu.async_copy(g_hbm.at[s_idx.at[e1]], rows_b, sem_b)
            pltpu.make_async_copy(g_hbm.at[s_idx.at[e0]], rows_a,
                                  sem_a).wait()
            pltpu.sync_copy(rows_a, acc.at[d_idx.at[e0]], add=True)

            @pl.when(e1 + 1 < _IDXG)
            def _():
                pltpu.async_copy(g_hbm.at[s_idx.at[e1 + 1]], rows_a, sem_a)

            pltpu.make_async_copy(g_hbm.at[s_idx.at[e1]], rows_b,
                                  sem_b).wait()
            pltpu.sync_copy(rows_b, acc.at[d_idx.at[e1]], add=True)
            return c2

        lax.fori_loop(0, _IDXG // 2, body, 0)
        return carry

    lax.fori_loop(0, cpt // _IDXG, group, 0)


def _make_agg_kernel(dh):
    cpt = E_PAD // (NS * CHUNK)

    @functools.partial(
        pl.kernel,
        out_type=jax.ShapeDtypeStruct((2 * NP, dh), jnp.float32),
        mesh=_mesh,
        scratch_types=[
            pltpu.VMEM_SHARED((NP, dh), jnp.float32),
            pltpu.VMEM((_IDXG, CHUNK), jnp.int32),
            pltpu.VMEM((_IDXG, CHUNK), jnp.int32),
            pltpu.VMEM((CHUNK, dh), jnp.float32),
            pltpu.VMEM((CHUNK, dh), jnp.float32),
            pltpu.SemaphoreType.DMA,
            pltpu.SemaphoreType.DMA,
        ],
    )
    def agg_kernel(g_hbm, src_hbm, dst_hbm, u_out, acc, s_idx, d_idx,
                   rows_a, rows_b, sem_a, sem_b):
        c = lax.axis_index("c")
        s = lax.axis_index("s")
        pltpu.sync_copy(g_hbm.at[pl.ds(c * NP + s * ROWS_T, ROWS_T)],
                        acc.at[pl.ds(s * ROWS_T, ROWS_T)])
        plsc.subcore_barrier()
        _edge_pipeline(g_hbm, acc, src_hbm, dst_hbm,
                       c * (cpt * NS) + s * cpt, s * cpt,
                       s_idx, d_idx, rows_a, rows_b, sem_a, sem_b, cpt)
        plsc.subcore_barrier()
        pltpu.sync_copy(acc.at[pl.ds(s * ROWS_T, ROWS_T)],
                        u_out.at[pl.ds(c * NP + s * ROWS_T, ROWS_T)])

    return agg_kernel


def _make_agg3_kernel():
    cpt = E_PAD // (NC * NS * CHUNK)

    @functools.partial(
        pl.kernel,
        out_type=jax.ShapeDtypeStruct((2 * NP, 128), jnp.float32),
        mesh=_mesh,
        scratch_types=[
            pltpu.VMEM_SHARED((NP, 128), jnp.float32),
            pltpu.VMEM((_IDXG, CHUNK), jnp.int32),
            pltpu.VMEM((_IDXG, CHUNK), jnp.int32),
            pltpu.VMEM((CHUNK, 128), jnp.float32),
            pltpu.VMEM((CHUNK, 128), jnp.float32),
            pltpu.SemaphoreType.DMA,
            pltpu.SemaphoreType.DMA,
        ],
    )
    def agg3_kernel(g_hbm, zeros_hbm, src_hbm, dst_hbm, u_out,
                    acc, s_idx, d_idx, rows_a, rows_b, sem_a, sem_b):
        c = lax.axis_index("c")
        s = lax.axis_index("s")
        pltpu.sync_copy(zeros_hbm.at[pl.ds(s * ROWS_T, ROWS_T)],
                        acc.at[pl.ds(s * ROWS_T, ROWS_T)])
        base = c * (E_PAD // (NC * CHUNK)) + s * cpt
        plsc.subcore_barrier()
        _edge_pipeline(g_hbm, acc, src_hbm, dst_hbm, base, base,
                       s_idx, d_idx, rows_a, rows_b, sem_a, sem_b, cpt)
        plsc.subcore_barrier()
        pltpu.sync_copy(acc.at[pl.ds(s * ROWS_T, ROWS_T)],
                        u_out.at[pl.ds(c * NP + s * ROWS_T, ROWS_T)])

    return agg3_kernel


_deg_kernel = _make_deg_kernel()
_agg128 = _make_agg_kernel(128)
_agg3 = _make_agg3_kernel()


_BM = 640
_GRID = NP // _BM


def _elu(v):
    return jnp.where(v > 0, v, jnp.exp(jnp.minimum(v, 0.0)) - 1.0)


def _tc1_body(x_ref, w_ref, deg0_ref, deg1_ref, g_ref, dinv_ref):
    deg = deg0_ref[:, 0] + deg1_ref[:, 0] + 1.0
    dinv = lax.rsqrt(deg)[:, None]
    dinv_ref[...] = jnp.broadcast_to(dinv, dinv_ref.shape)
    h = jnp.dot(x_ref[...], w_ref[...], preferred_element_type=jnp.float32)
    g = h * dinv
    g_ref[0] = g[:, :128]
    g_ref[1] = g[:, 128:]


def _tc2_body(u_ref, dinv_ref, b_ref, w_ref, x1_ref, g_ref):
    dinv = dinv_ref[:, :1]
    u = jnp.concatenate([u_ref[0], u_ref[1]], axis=1)
    x1 = _elu(dinv * u + b_ref[...])
    x1_ref[...] = x1
    g = jnp.dot(x1, w_ref[...], preferred_element_type=jnp.float32) * dinv
    g_ref[0] = g[:, :128]
    g_ref[1] = g[:, 128:]


def _tc3_body(u_ref, dinv_ref, b_ref, x1_ref, w_ref, g_ref):
    dinv = dinv_ref[:, :1]
    u = jnp.concatenate([u_ref[0], u_ref[1]], axis=1)
    x2 = _elu(dinv * u + b_ref[...] + x1_ref[...])
    g = jnp.dot(x2, w_ref[...], preferred_element_type=jnp.float32) * dinv
    g_ref[...] = jnp.concatenate([g, jnp.zeros_like(g)], axis=1)


def _tc4_body(u_ref, g_ref, dinv_ref, b_ref, o_ref):
    u = u_ref[0, :, :CLS] + u_ref[1, :, :CLS] + g_ref[:, :CLS]
    o_ref[...] = dinv_ref[:, :1] * u + b_ref[...]


def _row_spec(cols):
    return pl.BlockSpec((_BM, cols), lambda i: (i, 0))


def _half_spec(cols):
    return pl.BlockSpec((2, _BM, cols), lambda i: (0, i, 0))


def _full_spec(shape):
    return pl.BlockSpec(shape, lambda i: tuple(0 for _ in shape))


_tc1 = pl.pallas_call(
    _tc1_body,
    grid=(_GRID,),
    in_specs=[_row_spec(D), _full_spec((D, D)), _row_spec(128),
              pl.BlockSpec((_BM, 128), lambda i: (_GRID + i, 0))],
    out_specs=[_half_spec(128), _row_spec(8)],
    out_shape=[jax.ShapeDtypeStruct((2, NP, 128), jnp.float32),
               jax.ShapeDtypeStruct((NP, 8), jnp.float32)],
)

_tc2 = pl.pallas_call(
    _tc2_body,
    grid=(_GRID,),
    in_specs=[_half_spec(128), _row_spec(8), _full_spec((1, D)),
              _full_spec((D, D))],
    out_specs=[_row_spec(D), _half_spec(128)],
    out_shape=[jax.ShapeDtypeStruct((NP, D), jnp.float32),
               jax.ShapeDtypeStruct((2, NP, 128), jnp.float32)],
)

_tc3 = pl.pallas_call(
    _tc3_body,
    grid=(_GRID,),
    in_specs=[_half_spec(128), _row_spec(8), _full_spec((1, D)),
              _row_spec(D), _full_spec((D, CLS))],
    out_specs=_row_spec(128),
    out_shape=jax.ShapeDtypeStruct((NP, 128), jnp.float32),
)

_tc4 = pl.pallas_call(
    _tc4_body,
    grid=(_GRID,),
    in_specs=[_half_spec(128), _row_spec(128), _row_spec(8),
              _full_spec((1, CLS))],
    out_specs=_row_spec(CLS),
    out_shape=jax.ShapeDtypeStruct((NP, CLS), jnp.float32),
)


@jax.jit
def kernel(x, edge_index, batch, W1, b1, W2, b2, W3, b3):
    del batch
    src = edge_index[0]
    dst = edge_index[1]
    pad = E_PAD - E
    srcp = jnp.concatenate([src, jnp.zeros((pad,), jnp.int32)])
    dstp = jnp.concatenate([dst, jnp.full((pad,), N, jnp.int32)])
    src2 = jnp.stack([srcp, srcp + NP]).reshape(2 * (E_PAD // CHUNK), CHUNK)
    dstm = dstp.reshape(E_PAD // CHUNK, CHUNK)
    xp = jnp.pad(x, ((0, NP - N), (0, 0)))

    degf = _deg_kernel(dstm, jnp.zeros((NP, 128), jnp.float32),
                       jnp.ones((CHUNK, 128), jnp.float32))
    g1, dinvb = _tc1(xp, W1, degf, degf)
    u1 = _agg128(g1.reshape(2 * NP, 128), src2, dstm)
    x1, g2 = _tc2(u1.reshape(2, NP, 128), dinvb, b1.reshape(1, D), W2)
    u2 = _agg128(g2.reshape(2 * NP, 128), src2, dstm)
    g3 = _tc3(u2.reshape(2, NP, 128), dinvb, b2.reshape(1, D), x1, W3)
    u3 = _agg3(g3, jnp.zeros((NP, 128), jnp.float32),
               srcp.reshape(E_PAD // CHUNK, CHUNK), dstm)
    out = _tc4(u3.reshape(2, NP, 128), g3, dinvb, b3.reshape(1, CLS))
    return out[:N]


# speedup vs baseline: 1.1941x; 1.0541x over previous
import functools

import jax
import jax.numpy as jnp
from jax import lax
from jax.experimental import pallas as pl
from jax.experimental.pallas import tpu as pltpu
from jax.experimental.pallas import tpu_sc as plsc

N = 10000
NP = 10240
E = 160000
D = 256
CLS = 64
NC, NS = 2, 16
CHUNK = 128
E_PAD = 163840
ROWS_T = NP // NS

_mesh = plsc.VectorSubcoreMesh(core_axis_name="c", subcore_axis_name="s")


def _make_deg_kernel():
    cpt = E_PAD // (NC * NS * CHUNK)

    @functools.partial(
        pl.kernel,
        out_type=jax.ShapeDtypeStruct((2 * NP, 128), jnp.float32),
        mesh=_mesh,
        scratch_types=[
            pltpu.VMEM_SHARED((NP, 128), jnp.float32),
            pltpu.VMEM((cpt, CHUNK), jnp.int32),
            pltpu.VMEM((CHUNK, 128), jnp.float32),
        ],
    )
    def deg_kernel(dst_hbm, zeros_hbm, ones_hbm, deg_out, acc, d_idx, ones_v):
        c = lax.axis_index("c")
        s = lax.axis_index("s")
        pltpu.sync_copy(zeros_hbm.at[pl.ds(s * ROWS_T, ROWS_T)],
                        acc.at[pl.ds(s * ROWS_T, ROWS_T)])
        pltpu.sync_copy(ones_hbm, ones_v)
        base = c * (E_PAD // (NC * CHUNK)) + s * cpt
        pltpu.sync_copy(dst_hbm.at[pl.ds(base, cpt)], d_idx)
        plsc.subcore_barrier()

        def body(j, carry):
            pltpu.sync_copy(ones_v, acc.at[d_idx.at[j]], add=True)
            return carry

        lax.fori_loop(0, cpt, body, 0)
        plsc.subcore_barrier()
        pltpu.sync_copy(acc.at[pl.ds(s * ROWS_T, ROWS_T)],
                        deg_out.at[pl.ds(c * NP + s * ROWS_T, ROWS_T)])

    return deg_kernel


_IDXG = 40


def _edge_pipeline(g_hbm, acc, src_hbm, dst_hbm, sbase, dbase,
                   s_idx, d_idx, rows_a, rows_b, sem_a, sem_b, cpt):
    def group(gi, carry):
        pltpu.sync_copy(src_hbm.at[pl.ds(sbase + gi * _IDXG, _IDXG)], s_idx)
        pltpu.sync_copy(dst_hbm.at[pl.ds(dbase + gi * _IDXG, _IDXG)], d_idx)
        pltpu.async_copy(g_hbm.at[s_idx.at[0]], rows_a, sem_a)

        def body(i, c2):
            e0 = 2 * i
            e1 = e0 + 1
            pltpu.async_copy(g_hbm.at[s_idx.at[e1]], rows_b, sem_b)
            pltpu.make_async_copy(g_hbm.at[s_idx.at[e0]], rows_a,
                                  sem_a).wait()
            pltpu.sync_copy(rows_a, acc.at[d_idx.at[e0]], add=True)

            @pl.when(e1 + 1 < _IDXG)
            def _():
                pltpu.async_copy(g_hbm.at[s_idx.at[e1 + 1]], rows_a, sem_a)

            pltpu.make_async_copy(g_hbm.at[s_idx.at[e1]], rows_b,
                                  sem_b).wait()
            pltpu.sync_copy(rows_b, acc.at[d_idx.at[e1]], add=True)
            return c2

        lax.fori_loop(0, _IDXG // 2, body, 0)
        return carry

    lax.fori_loop(0, cpt // _IDXG, group, 0)


def _make_agg_kernel(dh):
    cpt = E_PAD // (NS * CHUNK)

    @functools.partial(
        pl.kernel,
        out_type=jax.ShapeDtypeStruct((2 * NP, dh), jnp.float32),
        mesh=_mesh,
        scratch_types=[
            pltpu.VMEM_SHARED((NP, dh), jnp.float32),
            pltpu.VMEM((_IDXG, CHUNK), jnp.int32),
            pltpu.VMEM((_IDXG, CHUNK), jnp.int32),
            pltpu.VMEM((CHUNK, dh), jnp.float32),
            pltpu.VMEM((CHUNK, dh), jnp.float32),
            pltpu.SemaphoreType.DMA,
            pltpu.SemaphoreType.DMA,
        ],
    )
    def agg_kernel(g_hbm, src_hbm, dst_hbm, u_out, acc, s_idx, d_idx,
                   rows_a, rows_b, sem_a, sem_b):
        c = lax.axis_index("c")
        s = lax.axis_index("s")
        pltpu.sync_copy(g_hbm.at[pl.ds(c * NP + s * ROWS_T, ROWS_T)],
                        acc.at[pl.ds(s * ROWS_T, ROWS_T)])
        plsc.subcore_barrier()
        _edge_pipeline(g_hbm, acc, src_hbm, dst_hbm,
                       c * (cpt * NS) + s * cpt, s * cpt,
                       s_idx, d_idx, rows_a, rows_b, sem_a, sem_b, cpt)
        plsc.subcore_barrier()
        pltpu.sync_copy(acc.at[pl.ds(s * ROWS_T, ROWS_T)],
                        u_out.at[pl.ds(c * NP + s * ROWS_T, ROWS_T)])

    return agg_kernel


def _make_agg3_kernel():
    cpt = E_PAD // (NC * NS * CHUNK)

    @functools.partial(
        pl.kernel,
        out_type=jax.ShapeDtypeStruct((2 * NP, 128), jnp.float32),
        mesh=_mesh,
        scratch_types=[
            pltpu.VMEM_SHARED((NP, 128), jnp.float32),
            pltpu.VMEM((_IDXG, CHUNK), jnp.int32),
            pltpu.VMEM((_IDXG, CHUNK), jnp.int32),
            pltpu.VMEM((CHUNK, 128), jnp.float32),
            pltpu.VMEM((CHUNK, 128), jnp.float32),
            pltpu.SemaphoreType.DMA,
            pltpu.SemaphoreType.DMA,
        ],
    )
    def agg3_kernel(g_hbm, zeros_hbm, src_hbm, dst_hbm, u_out,
                    acc, s_idx, d_idx, rows_a, rows_b, sem_a, sem_b):
        c = lax.axis_index("c")
        s = lax.axis_index("s")
        pltpu.sync_copy(zeros_hbm.at[pl.ds(s * ROWS_T, ROWS_T)],
                        acc.at[pl.ds(s * ROWS_T, ROWS_T)])
        base = c * (E_PAD // (NC * CHUNK)) + s * cpt
        plsc.subcore_barrier()
        _edge_pipeline(g_hbm, acc, src_hbm, dst_hbm, base, base,
                       s_idx, d_idx, rows_a, rows_b, sem_a, sem_b, cpt)
        plsc.subcore_barrier()
        pltpu.sync_copy(acc.at[pl.ds(s * ROWS_T, ROWS_T)],
                        u_out.at[pl.ds(c * NP + s * ROWS_T, ROWS_T)])

    return agg3_kernel


_deg_kernel = _make_deg_kernel()
_agg128 = _make_agg_kernel(128)
_agg3 = _make_agg3_kernel()


_BM = 640
_GRID = NP // _BM


def _elu(v):
    return jnp.where(v > 0, v, jnp.exp(jnp.minimum(v, 0.0)) - 1.0)


def _tc1_body(x_ref, w_ref, deg0_ref, deg1_ref, g_ref, dinv_ref):
    deg = deg0_ref[:, 0] + deg1_ref[:, 0] + 1.0
    dinv = lax.rsqrt(deg)[:, None]
    dinv_ref[...] = jnp.broadcast_to(dinv, dinv_ref.shape)
    h = jnp.dot(x_ref[...], w_ref[...], preferred_element_type=jnp.float32)
    g = h * dinv
    g_ref[0] = g[:, :128]
    g_ref[1] = g[:, 128:]


def _tc2_body(u_ref, dinv_ref, b_ref, w_ref, x1_ref, g_ref):
    dinv = dinv_ref[:, :1]
    u = jnp.concatenate([u_ref[0], u_ref[1]], axis=1)
    x1 = _elu(dinv * u + b_ref[...])
    x1_ref[...] = x1
    g = jnp.dot(x1, w_ref[...], preferred_element_type=jnp.float32) * dinv
    g_ref[0] = g[:, :128]
    g_ref[1] = g[:, 128:]


def _tc3_body(u_ref, dinv_ref, b_ref, x1_ref, w_ref, g_ref):
    dinv = dinv_ref[:, :1]
    u = jnp.concatenate([u_ref[0], u_ref[1]], axis=1)
    x2 = _elu(dinv * u + b_ref[...] + x1_ref[...])
    g = jnp.dot(x2, w_ref[...], preferred_element_type=jnp.float32) * dinv
    g_ref[...] = jnp.concatenate([g, jnp.zeros_like(g)], axis=1)


def _tc4_body(u_ref, g_ref, dinv_ref, b_ref, o_ref):
    u = u_ref[0, :, :CLS] + u_ref[1, :, :CLS] + g_ref[:, :CLS]
    o_ref[...] = dinv_ref[:, :1] * u + b_ref[...]


def _row_spec(cols):
    return pl.BlockSpec((_BM, cols), lambda i: (i, 0))


def _half_spec(cols):
    return pl.BlockSpec((2, _BM, cols), lambda i: (0, i, 0))


def _full_spec(shape):
    return pl.BlockSpec(shape, lambda i: tuple(0 for _ in shape))


_tc1 = pl.pallas_call(
    _tc1_body,
    grid=(_GRID,),
    in_specs=[_row_spec(D), _full_spec((D, D)), _row_spec(128),
              pl.BlockSpec((_BM, 128), lambda i: (_GRID + i, 0))],
    out_specs=[_half_spec(128), _row_spec(8)],
    out_shape=[jax.ShapeDtypeStruct((2, NP, 128), jnp.float32),
               jax.ShapeDtypeStruct((NP, 8), jnp.float32)],
)

_tc2 = pl.pallas_call(
    _tc2_body,
    grid=(_GRID,),
    in_specs=[_half_spec(128), _row_spec(8), _full_spec((1, D)),
              _full_spec((D, D))],
    out_specs=[_row_spec(D), _half_spec(128)],
    out_shape=[jax.ShapeDtypeStruct((NP, D), jnp.float32),
               jax.ShapeDtypeStruct((2, NP, 128), jnp.float32)],
)

_tc3 = pl.pallas_call(
    _tc3_body,
    grid=(_GRID,),
    in_specs=[_half_spec(128), _row_spec(8), _full_spec((1, D)),
              _row_spec(D), _full_spec((D, CLS))],
    out_specs=_row_spec(128),
    out_shape=jax.ShapeDtypeStruct((NP, 128), jnp.float32),
)

_tc4 = pl.pallas_call(
    _tc4_body,
    grid=(_GRID,),
    in_specs=[_half_spec(128), _row_spec(128), _row_spec(8),
              _full_spec((1, CLS))],
    out_specs=_row_spec(CLS),
    out_shape=jax.ShapeDtypeStruct((NP, CLS), jnp.float32),
)


@jax.jit
def kernel(x, edge_index, batch, W1, b1, W2, b2, W3, b3):
    del batch
    src = edge_index[0]
    dst = edge_index[1]
    pad = E_PAD - E
    srcp = jnp.concatenate([src, jnp.zeros((pad,), jnp.int32)])
    dstp = jnp.concatenate([dst, jnp.full((pad,), N, jnp.int32)])
    src2 = jnp.stack([srcp, srcp + NP]).reshape(2 * (E_PAD // CHUNK), CHUNK)
    dstm = dstp.reshape(E_PAD // CHUNK, CHUNK)
    xp = jnp.pad(x, ((0, NP - N), (0, 0)))

    degf = _deg_kernel(dstm, jnp.zeros((NP, 128), jnp.float32),
                       jnp.ones((CHUNK, 128), jnp.float32))
    g1, dinvb = _tc1(xp, W1, degf, degf)
    u1 = _agg128(g1.reshape(2 * NP, 128), src2, dstm)
    x1, g2 = _tc2(u1.reshape(2, NP, 128), dinvb, b1.reshape(1, D), W2)
    u2 = _agg128(g2.reshape(2 * NP, 128), src2, dstm)
    g3 = _tc3(u2.reshape(2, NP, 128), dinvb, b2.reshape(1, D), x1, W3)
    u3 = _agg3(g3, jnp.zeros((NP, 128), jnp.float32),
               srcp.reshape(E_PAD // CHUNK, CHUNK), dstm)
    out = _tc4(u3.reshape(2, NP, 128), g3, dinvb, b3.reshape(1, CLS))
    return out[:N]
